# baseline (speedup 1.0000x reference)
import functools

import jax
import jax.numpy as jnp
from jax import lax
from jax.experimental import pallas as pl
from jax.experimental.pallas import tpu as pltpu
from jax.experimental.pallas import tpu_sc as plsc

MD = 768
NE = 16
NT = 1024
HD = 2 * MD
NP = 2 * NT
BS = 512
NB = NP // BS + NE
NR = NB * BS

MD2 = MD // 2

NC = 2
NS = 16
NW = NC * NS
CH = NP // NW


def _pack(a, b):
    au = lax.bitcast_convert_type(
        a.astype(jnp.bfloat16).astype(jnp.float32), jnp.uint32)
    bu = lax.bitcast_convert_type(
        b.astype(jnp.bfloat16).astype(jnp.float32), jnp.uint32)
    w = (au >> 16) | (bu & jnp.uint32(0xFFFF0000))
    return lax.bitcast_convert_type(w, jnp.float32)


def _unpack(p):
    u = lax.bitcast_convert_type(p, jnp.uint32)
    a = lax.bitcast_convert_type(u << 16, jnp.float32)
    b = lax.bitcast_convert_type(u & jnp.uint32(0xFFFF0000), jnp.float32)
    return a, b


def _gate_body(x_ref, wg_ref, logits_ref, topidx_ref, wts_ref, pos_ref,
               be_ref, nb_ref, xp_ref):
    x = x_ref[...]
    xp_ref[...] = _pack(x[:, :MD2], x[:, MD2:])
    logits = jnp.dot(x, wg_ref[...], preferred_element_type=jnp.float32)
    logits_ref[...] = logits
    col = lax.broadcasted_iota(jnp.int32, (NT, NE), 1)
    m1 = jnp.max(logits, axis=1, keepdims=True)
    a1 = jnp.min(jnp.where(logits == m1, col, NE), axis=1, keepdims=True)
    masked = jnp.where(col == a1, -jnp.inf, logits)
    m2 = jnp.max(masked, axis=1, keepdims=True)
    a2 = jnp.min(jnp.where(masked == m2, col, NE), axis=1, keepdims=True)
    e2 = jnp.exp(m2 - m1)
    s = 1.0 + e2
    topidx_ref[...] = jnp.concatenate([a1, a2], axis=1)
    wts_ref[...] = jnp.concatenate([1.0 / s, e2 / s], axis=1)

    onehot = jnp.concatenate([col == a1, col == a2], axis=0)
    onehot = onehot.astype(jnp.float32)
    CS = 256
    r = lax.broadcasted_iota(jnp.int32, (CS, CS), 0)
    c = lax.broadcasted_iota(jnp.int32, (CS, CS), 1)
    ltri = (r >= c).astype(jnp.float32)
    parts = []
    carry = jnp.zeros((1, NE), jnp.float32)
    for ci in range(NP // CS):
        rc = jnp.dot(ltri, onehot[ci * CS:(ci + 1) * CS, :],
                     preferred_element_type=jnp.float32,
                     precision=lax.Precision.DEFAULT) + carry
        parts.append(rc)
        carry = rc[CS - 1:CS, :]
    csum = jnp.concatenate(parts, axis=0)
    counts = carry
    padded = jnp.floor((counts + (BS - 1)) / BS) * BS
    er = lax.broadcasted_iota(jnp.int32, (NE, NE), 0)
    ec = lax.broadcasted_iota(jnp.int32, (NE, NE), 1)
    stri = (er < ec).astype(jnp.float32)
    offs = jnp.dot(padded, stri, preferred_element_type=jnp.float32)
    total = jnp.sum(padded, axis=1, keepdims=True)
    rank = jnp.sum(onehot * csum, axis=1, keepdims=True)
    roff = jnp.sum(onehot * offs, axis=1, keepdims=True)
    pos_ref[...] = (rank - 1.0 + roff).astype(jnp.int32)
    brow = lax.broadcasted_iota(jnp.int32, (NB, 1), 0).astype(jnp.float32) * BS
    pos_b = jnp.minimum(brow, total - BS)
    be_ref[...] = jnp.sum((offs <= pos_b).astype(jnp.int32), axis=1,
                          keepdims=True) - 1
    nb_ref[...] = (total / BS).astype(jnp.int32)


def _gate(x, w_gate):
    return pl.pallas_call(
        _gate_body,
        out_shape=[
            jax.ShapeDtypeStruct((NT, NE), jnp.float32),
            jax.ShapeDtypeStruct((NT, 2), jnp.int32),
            jax.ShapeDtypeStruct((NT, 2), jnp.float32),
            jax.ShapeDtypeStruct((NP, 1), jnp.int32),
            jax.ShapeDtypeStruct((NB, 1), jnp.int32),
            jax.ShapeDtypeStruct((1, 1), jnp.int32),
            jax.ShapeDtypeStruct((NT, MD2), jnp.float32),
        ],
    )(x, w_gate)


def _dispatch_body(x_hbm, p_hbm, xd_hbm, idx_v, rows_v, sem):
    wid = lax.axis_index("s") * NC + lax.axis_index("c")
    base = wid * CH
    tbase = lax.rem(base, NT)
    pltpu.sync_copy(p_hbm.at[pl.ds(base, CH)], idx_v)
    pltpu.sync_copy(x_hbm.at[pl.ds(tbase, CH)], rows_v)
    pltpu.async_copy(rows_v, xd_hbm.at[idx_v], sem).wait()


def _sc_mesh():
    return plsc.VectorSubcoreMesh(core_axis_name="c", subcore_axis_name="s",
                                  num_cores=NC, num_subcores=NS)


def _dispatch(x, pflat):
    return pl.kernel(
        _dispatch_body,
        out_type=jax.ShapeDtypeStruct((NR, MD2), jnp.float32),
        mesh=_sc_mesh(),
        scratch_types=[
            pltpu.VMEM((CH,), jnp.int32),
            pltpu.VMEM((CH, MD2), jnp.float32),
            pltpu.SemaphoreType.DMA,
        ],
    )(x, pflat)


def _experts_body(be_ref, nb_ref, xb_ref, w1_ref, b1_ref, w2_ref, b2_ref,
                  o_ref):
    b = pl.program_id(0)

    @pl.when(b < nb_ref[0])
    def _():
        xa, xb = _unpack(xb_ref[...])
        xf = jnp.concatenate([xa, xb], axis=1)
        h = jnp.dot(xf, w1_ref[0], preferred_element_type=jnp.float32,
                    precision=lax.Precision.DEFAULT) + b1_ref[0]
        h = jnp.maximum(h, 0.0)
        y = jnp.dot(h, w2_ref[0], preferred_element_type=jnp.float32,
                    precision=lax.Precision.DEFAULT) + b2_ref[0]
        o_ref[...] = _pack(y[:, :MD2], y[:, MD2:])


def _experts(be, nb, xd, W1, b1, W2, b2):
    grid_spec = pltpu.PrefetchScalarGridSpec(
        num_scalar_prefetch=2,
        grid=(NB,),
        in_specs=[
            pl.BlockSpec((BS, MD2),
                         lambda b, be, nb: (jnp.minimum(b, nb[0] - 1), 0)),
            pl.BlockSpec((1, MD, HD), lambda b, be, nb: (be[b], 0, 0)),
            pl.BlockSpec((1, 1, HD), lambda b, be, nb: (be[b], 0, 0)),
            pl.BlockSpec((1, HD, MD), lambda b, be, nb: (be[b], 0, 0)),
            pl.BlockSpec((1, 1, MD), lambda b, be, nb: (be[b], 0, 0)),
        ],
        out_specs=pl.BlockSpec(
            (BS, MD2), lambda b, be, nb: (jnp.minimum(b, nb[0] - 1), 0)),
    )
    return pl.pallas_call(
        _experts_body,
        grid_spec=grid_spec,
        out_shape=jax.ShapeDtypeStruct((NR, MD2), jnp.float32),
    )(be, nb, xd, W1, b1.reshape(NE, 1, HD), W2, b2.reshape(NE, 1, MD))


def _gather_body(yd_hbm, p_hbm, g_hbm, idx_v, rows_v, sem):
    wid = lax.axis_index("s") * NC + lax.axis_index("c")
    base = wid * CH
    pltpu.sync_copy(p_hbm.at[pl.ds(base, CH)], idx_v)
    pltpu.async_copy(yd_hbm.at[idx_v], rows_v, sem).wait()
    pltpu.sync_copy(rows_v, g_hbm.at[pl.ds(base, CH)])


def _gather(yd, pflat):
    return pl.kernel(
        _gather_body,
        out_type=jax.ShapeDtypeStruct((NP, MD2), jnp.float32),
        mesh=_sc_mesh(),
        scratch_types=[
            pltpu.VMEM((CH,), jnp.int32),
            pltpu.VMEM((CH, MD2), jnp.float32),
            pltpu.SemaphoreType.DMA,
        ],
    )(yd, pflat)


def _combine_body(g_ref, w_ref, o_ref):
    ga, gb = _unpack(g_ref[...])
    g = jnp.concatenate([ga, gb], axis=1)
    o_ref[...] = w_ref[:, 0:1] * g[:NT, :] + w_ref[:, 1:2] * g[NT:, :]


def _combine(g, wts):
    return pl.pallas_call(
        _combine_body,
        out_shape=jax.ShapeDtypeStruct((NT, MD), jnp.float32),
    )(g, wts)


def kernel(x, w_gate, W1, b1, W2, b2):
    logits, top_idx, wts, pos, be, nb, xp = _gate(x, w_gate)
    pflat = pos.reshape(NP)
    xd = _dispatch(xp, pflat)
    yd = _experts(be.reshape(NB), nb.reshape(1), xd, W1, b1, W2, b2)
    g = _gather(yd, pflat)
    out = _combine(g, wts)
    return out, logits, top_idx


# speedup vs baseline: 1.1165x; 1.1165x over previous
import functools

import jax
import jax.numpy as jnp
from jax import lax
from jax.experimental import pallas as pl
from jax.experimental.pallas import tpu as pltpu
from jax.experimental.pallas import tpu_sc as plsc

MD = 768
NE = 16
NT = 1024
HD = 2 * MD
NP = 2 * NT
BS = 256
NB = NP // BS + NE
NR = NB * BS

MD2 = MD // 2

NC = 2
NS = 16
NW = NC * NS
CH = NP // NW


def _pack(a, b):
    au = lax.bitcast_convert_type(
        a.astype(jnp.bfloat16).astype(jnp.float32), jnp.uint32)
    bu = lax.bitcast_convert_type(
        b.astype(jnp.bfloat16).astype(jnp.float32), jnp.uint32)
    w = (au >> 16) | (bu & jnp.uint32(0xFFFF0000))
    return lax.bitcast_convert_type(w, jnp.float32)


def _unpack(p):
    u = lax.bitcast_convert_type(p, jnp.uint32)
    a = lax.bitcast_convert_type(u << 16, jnp.float32)
    b = lax.bitcast_convert_type(u & jnp.uint32(0xFFFF0000), jnp.float32)
    return a, b


def _gate_body(x_ref, wg_ref, logits_ref, topidx_ref, wts_ref, pos_ref,
               be_ref, nb_ref, xp_ref):
    x = x_ref[...]
    xp_ref[...] = _pack(x[:, :MD2], x[:, MD2:])
    logits = jnp.dot(x, wg_ref[...], preferred_element_type=jnp.float32)
    logits_ref[...] = logits
    col = lax.broadcasted_iota(jnp.int32, (NT, NE), 1)
    m1 = jnp.max(logits, axis=1, keepdims=True)
    a1 = jnp.min(jnp.where(logits == m1, col, NE), axis=1, keepdims=True)
    masked = jnp.where(col == a1, -jnp.inf, logits)
    m2 = jnp.max(masked, axis=1, keepdims=True)
    a2 = jnp.min(jnp.where(masked == m2, col, NE), axis=1, keepdims=True)
    e2 = jnp.exp(m2 - m1)
    s = 1.0 + e2
    topidx_ref[...] = jnp.concatenate([a1, a2], axis=1)
    wts_ref[...] = jnp.concatenate([1.0 / s, e2 / s], axis=1)

    onehot = jnp.concatenate([col == a1, col == a2], axis=0)
    onehot = onehot.astype(jnp.float32)
    CS = 256
    r = lax.broadcasted_iota(jnp.int32, (CS, CS), 0)
    c = lax.broadcasted_iota(jnp.int32, (CS, CS), 1)
    ltri = (r >= c).astype(jnp.float32)
    parts = []
    carry = jnp.zeros((1, NE), jnp.float32)
    for ci in range(NP // CS):
        rc = jnp.dot(ltri, onehot[ci * CS:(ci + 1) * CS, :],
                     preferred_element_type=jnp.float32,
                     precision=lax.Precision.DEFAULT) + carry
        parts.append(rc)
        carry = rc[CS - 1:CS, :]
    csum = jnp.concatenate(parts, axis=0)
    counts = carry
    padded = jnp.floor((counts + (BS - 1)) / BS) * BS
    er = lax.broadcasted_iota(jnp.int32, (NE, NE), 0)
    ec = lax.broadcasted_iota(jnp.int32, (NE, NE), 1)
    stri = (er < ec).astype(jnp.float32)
    offs = jnp.dot(padded, stri, preferred_element_type=jnp.float32)
    total = jnp.sum(padded, axis=1, keepdims=True)
    rank = jnp.sum(onehot * csum, axis=1, keepdims=True)
    roff = jnp.sum(onehot * offs, axis=1, keepdims=True)
    pos_ref[...] = (rank - 1.0 + roff).astype(jnp.int32)
    brow = lax.broadcasted_iota(jnp.int32, (NB, 1), 0).astype(jnp.float32) * BS
    pos_b = jnp.minimum(brow, total - BS)
    be_ref[...] = jnp.sum((offs <= pos_b).astype(jnp.int32), axis=1,
                          keepdims=True) - 1
    nb_ref[...] = (total / BS).astype(jnp.int32)


def _gate(x, w_gate):
    return pl.pallas_call(
        _gate_body,
        out_shape=[
            jax.ShapeDtypeStruct((NT, NE), jnp.float32),
            jax.ShapeDtypeStruct((NT, 2), jnp.int32),
            jax.ShapeDtypeStruct((NT, 2), jnp.float32),
            jax.ShapeDtypeStruct((NP, 1), jnp.int32),
            jax.ShapeDtypeStruct((NB, 1), jnp.int32),
            jax.ShapeDtypeStruct((1, 1), jnp.int32),
            jax.ShapeDtypeStruct((NT, MD2), jnp.float32),
        ],
    )(x, w_gate)


def _dispatch_body(x_hbm, p_hbm, xd_hbm, idx_v, rows_v, sem):
    wid = lax.axis_index("s") * NC + lax.axis_index("c")
    base = wid * CH
    tbase = lax.rem(base, NT)
    pltpu.sync_copy(p_hbm.at[pl.ds(base, CH)], idx_v)
    pltpu.sync_copy(x_hbm.at[pl.ds(tbase, CH)], rows_v)
    pltpu.async_copy(rows_v, xd_hbm.at[idx_v], sem).wait()


def _sc_mesh():
    return plsc.VectorSubcoreMesh(core_axis_name="c", subcore_axis_name="s",
                                  num_cores=NC, num_subcores=NS)


def _dispatch(x, pflat):
    return pl.kernel(
        _dispatch_body,
        out_type=jax.ShapeDtypeStruct((NR, MD2), jnp.float32),
        mesh=_sc_mesh(),
        scratch_types=[
            pltpu.VMEM((CH,), jnp.int32),
            pltpu.VMEM((CH, MD2), jnp.float32),
            pltpu.SemaphoreType.DMA,
        ],
    )(x, pflat)


def _experts_body(be_ref, nb_ref, xb_ref, w1_ref, b1_ref, w2_ref, b2_ref,
                  o_ref):
    b = pl.program_id(0)

    @pl.when(b < nb_ref[0])
    def _():
        xa, xb = _unpack(xb_ref[...])
        xf = jnp.concatenate([xa, xb], axis=1)
        h = jnp.dot(xf, w1_ref[0], preferred_element_type=jnp.float32,
                    precision=lax.Precision.DEFAULT) + b1_ref[0]
        h = jnp.maximum(h, 0.0)
        y = jnp.dot(h, w2_ref[0], preferred_element_type=jnp.float32,
                    precision=lax.Precision.DEFAULT) + b2_ref[0]
        o_ref[...] = _pack(y[:, :MD2], y[:, MD2:])


def _experts(be, nb, xd, W1, b1, W2, b2):
    grid_spec = pltpu.PrefetchScalarGridSpec(
        num_scalar_prefetch=2,
        grid=(NB,),
        in_specs=[
            pl.BlockSpec((BS, MD2),
                         lambda b, be, nb: (jnp.minimum(b, nb[0] - 1), 0)),
            pl.BlockSpec((1, MD, HD), lambda b, be, nb: (be[b], 0, 0)),
            pl.BlockSpec((1, 1, HD), lambda b, be, nb: (be[b], 0, 0)),
            pl.BlockSpec((1, HD, MD), lambda b, be, nb: (be[b], 0, 0)),
            pl.BlockSpec((1, 1, MD), lambda b, be, nb: (be[b], 0, 0)),
        ],
        out_specs=pl.BlockSpec(
            (BS, MD2), lambda b, be, nb: (jnp.minimum(b, nb[0] - 1), 0)),
    )
    return pl.pallas_call(
        _experts_body,
        grid_spec=grid_spec,
        out_shape=jax.ShapeDtypeStruct((NR, MD2), jnp.float32),
    )(be, nb, xd, W1, b1.reshape(NE, 1, HD), W2, b2.reshape(NE, 1, MD))


def _gather_body(yd_hbm, p_hbm, g_hbm, idx_v, rows_v, sem):
    wid = lax.axis_index("s") * NC + lax.axis_index("c")
    base = wid * CH
    pltpu.sync_copy(p_hbm.at[pl.ds(base, CH)], idx_v)
    pltpu.async_copy(yd_hbm.at[idx_v], rows_v, sem).wait()
    pltpu.sync_copy(rows_v, g_hbm.at[pl.ds(base, CH)])


def _gather(yd, pflat):
    return pl.kernel(
        _gather_body,
        out_type=jax.ShapeDtypeStruct((NP, MD2), jnp.float32),
        mesh=_sc_mesh(),
        scratch_types=[
            pltpu.VMEM((CH,), jnp.int32),
            pltpu.VMEM((CH, MD2), jnp.float32),
            pltpu.SemaphoreType.DMA,
        ],
    )(yd, pflat)


def _combine_body(g_ref, w_ref, o_ref):
    ga, gb = _unpack(g_ref[...])
    g = jnp.concatenate([ga, gb], axis=1)
    o_ref[...] = w_ref[:, 0:1] * g[:NT, :] + w_ref[:, 1:2] * g[NT:, :]


def _combine(g, wts):
    return pl.pallas_call(
        _combine_body,
        out_shape=jax.ShapeDtypeStruct((NT, MD), jnp.float32),
    )(g, wts)


def kernel(x, w_gate, W1, b1, W2, b2):
    logits, top_idx, wts, pos, be, nb, xp = _gate(x, w_gate)
    pflat = pos.reshape(NP)
    xd = _dispatch(xp, pflat)
    yd = _experts(be.reshape(NB), nb.reshape(1), xd, W1, b1, W2, b2)
    g = _gather(yd, pflat)
    out = _combine(g, wts)
    return out, logits, top_idx


# speedup vs baseline: 1.1556x; 1.0350x over previous
import functools

import jax
import jax.numpy as jnp
from jax import lax
from jax.experimental import pallas as pl
from jax.experimental.pallas import tpu as pltpu
from jax.experimental.pallas import tpu_sc as plsc

MD = 768
NE = 16
NT = 1024
HD = 2 * MD
NP = 2 * NT
BS = 256
NB = NP // BS + NE
NR = NB * BS

MD2 = MD // 2

NC = 2
NS = 16
NW = NC * NS
CH = NP // NW


def _pack(a, b):
    au = lax.bitcast_convert_type(
        a.astype(jnp.bfloat16).astype(jnp.float32), jnp.uint32)
    bu = lax.bitcast_convert_type(
        b.astype(jnp.bfloat16).astype(jnp.float32), jnp.uint32)
    w = (au >> 16) | (bu & jnp.uint32(0xFFFF0000))
    return lax.bitcast_convert_type(w, jnp.float32)


def _unpack(p):
    u = lax.bitcast_convert_type(p, jnp.uint32)
    a = lax.bitcast_convert_type(u << 16, jnp.float32)
    b = lax.bitcast_convert_type(u & jnp.uint32(0xFFFF0000), jnp.float32)
    return a, b


def _gate_body(x_ref, wg_ref, logits_ref, topidx_ref, wts_ref, pos_ref,
               be_ref, nb_ref, xp_ref):
    x = x_ref[...]
    xp_ref[...] = _pack(x[:, :MD2], x[:, MD2:])
    logits = jnp.dot(x, wg_ref[...], preferred_element_type=jnp.float32)
    logits_ref[...] = logits
    col = lax.broadcasted_iota(jnp.int32, (NT, NE), 1)
    m1 = jnp.max(logits, axis=1, keepdims=True)
    a1 = jnp.min(jnp.where(logits == m1, col, NE), axis=1, keepdims=True)
    masked = jnp.where(col == a1, -jnp.inf, logits)
    m2 = jnp.max(masked, axis=1, keepdims=True)
    a2 = jnp.min(jnp.where(masked == m2, col, NE), axis=1, keepdims=True)
    e2 = jnp.exp(m2 - m1)
    s = 1.0 + e2
    topidx_ref[...] = jnp.concatenate([a1, a2], axis=1)
    wts_ref[...] = jnp.concatenate([1.0 / s, e2 / s], axis=1)

    onehot = jnp.concatenate([col == a1, col == a2], axis=0)
    onehot = onehot.astype(jnp.float32)
    CS = 256
    r = lax.broadcasted_iota(jnp.int32, (CS, CS), 0)
    c = lax.broadcasted_iota(jnp.int32, (CS, CS), 1)
    ltri = (r >= c).astype(jnp.float32)
    parts = []
    carry = jnp.zeros((1, NE), jnp.float32)
    for ci in range(NP // CS):
        rc = jnp.dot(ltri, onehot[ci * CS:(ci + 1) * CS, :],
                     preferred_element_type=jnp.float32,
                     precision=lax.Precision.DEFAULT) + carry
        parts.append(rc)
        carry = rc[CS - 1:CS, :]
    csum = jnp.concatenate(parts, axis=0)
    counts = carry
    padded = jnp.floor((counts + (BS - 1)) / BS) * BS
    er = lax.broadcasted_iota(jnp.int32, (NE, NE), 0)
    ec = lax.broadcasted_iota(jnp.int32, (NE, NE), 1)
    stri = (er < ec).astype(jnp.float32)
    offs = jnp.dot(padded, stri, preferred_element_type=jnp.float32)
    total = jnp.sum(padded, axis=1, keepdims=True)
    rank = jnp.sum(onehot * csum, axis=1, keepdims=True)
    roff = jnp.sum(onehot * offs, axis=1, keepdims=True)
    posv = (rank - 1.0 + roff).astype(jnp.int32)
    pos_ref[...] = posv.reshape(NP // 128, 128)
    brow = lax.broadcasted_iota(jnp.int32, (NB, 1), 0).astype(jnp.float32) * BS
    pos_b = jnp.minimum(brow, total - BS)
    be_ref[...] = jnp.sum((offs <= pos_b).astype(jnp.int32), axis=1,
                          keepdims=True) - 1
    nb_ref[...] = (total / BS).astype(jnp.int32)


def _gate(x, w_gate):
    return pl.pallas_call(
        _gate_body,
        out_shape=[
            jax.ShapeDtypeStruct((NT, NE), jnp.float32),
            jax.ShapeDtypeStruct((NT, 2), jnp.int32),
            jax.ShapeDtypeStruct((NT, 2), jnp.float32),
            jax.ShapeDtypeStruct((NP // 128, 128), jnp.int32),
            jax.ShapeDtypeStruct((NB, 1), jnp.int32),
            jax.ShapeDtypeStruct((1, 1), jnp.int32),
            jax.ShapeDtypeStruct((NT, MD2), jnp.float32),
        ],
    )(x, w_gate)


def _dispatch_body(x_hbm, p_hbm, xd_hbm, idx_v, rows_v, sem):
    wid = lax.axis_index("s") * NC + lax.axis_index("c")
    base = wid * CH
    tbase = lax.rem(base, NT)
    pltpu.sync_copy(p_hbm.at[pl.ds(base, CH)], idx_v)
    pltpu.sync_copy(x_hbm.at[pl.ds(tbase, CH)], rows_v)
    pltpu.async_copy(rows_v, xd_hbm.at[idx_v], sem).wait()


def _sc_mesh():
    return plsc.VectorSubcoreMesh(core_axis_name="c", subcore_axis_name="s",
                                  num_cores=NC, num_subcores=NS)


def _dispatch(x, pflat):
    return pl.kernel(
        _dispatch_body,
        out_type=jax.ShapeDtypeStruct((NR, MD2), jnp.float32),
        mesh=_sc_mesh(),
        scratch_types=[
            pltpu.VMEM((CH,), jnp.int32),
            pltpu.VMEM((CH, MD2), jnp.float32),
            pltpu.SemaphoreType.DMA,
        ],
    )(x, pflat)


def _experts_body(be_ref, nb_ref, xb_ref, w1_ref, b1_ref, w2_ref, b2_ref,
                  o_ref):
    b = pl.program_id(0)

    @pl.when(b < nb_ref[0])
    def _():
        xa, xb = _unpack(xb_ref[...])
        xf = jnp.concatenate([xa, xb], axis=1)
        e = be_ref[b]
        h = (jnp.dot(xf, w1_ref[0], preferred_element_type=jnp.float32,
                     precision=lax.Precision.DEFAULT)
             + b1_ref[pl.ds(e, 1), :])
        h = jnp.maximum(h, 0.0)
        y = (jnp.dot(h, w2_ref[0], preferred_element_type=jnp.float32,
                     precision=lax.Precision.DEFAULT)
             + b2_ref[pl.ds(e, 1), :])
        o_ref[...] = _pack(y[:, :MD2], y[:, MD2:])


def _experts(be, nb, xd, W1, b1, W2, b2):
    grid_spec = pltpu.PrefetchScalarGridSpec(
        num_scalar_prefetch=2,
        grid=(NB,),
        in_specs=[
            pl.BlockSpec((BS, MD2),
                         lambda b, be, nb: (jnp.minimum(b, nb[0] - 1), 0)),
            pl.BlockSpec((1, MD, HD), lambda b, be, nb: (be[b], 0, 0)),
            pl.BlockSpec((NE, HD), lambda b, be, nb: (0, 0)),
            pl.BlockSpec((1, HD, MD), lambda b, be, nb: (be[b], 0, 0)),
            pl.BlockSpec((NE, MD), lambda b, be, nb: (0, 0)),
        ],
        out_specs=pl.BlockSpec(
            (BS, MD2), lambda b, be, nb: (jnp.minimum(b, nb[0] - 1), 0)),
    )
    return pl.pallas_call(
        _experts_body,
        grid_spec=grid_spec,
        out_shape=jax.ShapeDtypeStruct((NR, MD2), jnp.float32),
    )(be, nb, xd, W1, b1, W2, b2)


def _gather_body(yd_hbm, p_hbm, g_hbm, idx_v, rows_v, sem):
    wid = lax.axis_index("s") * NC + lax.axis_index("c")
    base = wid * CH
    pltpu.sync_copy(p_hbm.at[pl.ds(base, CH)], idx_v)
    pltpu.async_copy(yd_hbm.at[idx_v], rows_v, sem).wait()
    pltpu.sync_copy(rows_v, g_hbm.at[pl.ds(base, CH)])


def _gather(yd, pflat):
    return pl.kernel(
        _gather_body,
        out_type=jax.ShapeDtypeStruct((NP, MD2), jnp.float32),
        mesh=_sc_mesh(),
        scratch_types=[
            pltpu.VMEM((CH,), jnp.int32),
            pltpu.VMEM((CH, MD2), jnp.float32),
            pltpu.SemaphoreType.DMA,
        ],
    )(yd, pflat)


def _combine_body(g_ref, w_ref, o_ref):
    ga, gb = _unpack(g_ref[...])
    g = jnp.concatenate([ga, gb], axis=1)
    o_ref[...] = w_ref[:, 0:1] * g[:NT, :] + w_ref[:, 1:2] * g[NT:, :]


def _combine(g, wts):
    return pl.pallas_call(
        _combine_body,
        out_shape=jax.ShapeDtypeStruct((NT, MD), jnp.float32),
    )(g, wts)


def kernel(x, w_gate, W1, b1, W2, b2):
    logits, top_idx, wts, pos, be, nb, xp = _gate(x, w_gate)
    pflat = pos.reshape(NP)
    xd = _dispatch(xp, pflat)
    yd = _experts(be.reshape(NB), nb.reshape(1), xd, W1, b1, W2, b2)
    g = _gather(yd, pflat)
    out = _combine(g, wts)
    return out, logits, top_idx
